# baseline (speedup 1.0000x reference)
import functools

import jax
import jax.numpy as jnp
from jax import lax
from jax.experimental import pallas as pl
from jax.experimental.pallas import tpu as pltpu
from jax.experimental.pallas import tpu_sc as plsc

H = 16
G = 64
NC = 2
NS = 16
NW = NC * NS
CH = 128
GS = 8

TE = 4096


def _lin_relu_fold_body(x_ref, w_ref, b_ref, o_ref):
    t, kdim = x_ref.shape
    x3 = x_ref[...].reshape(t // 8, 8, kdim)
    w = w_ref[...]
    b = b_ref[...]
    outs = []
    for p in range(8):
        outs.append(jax.nn.relu(
            jnp.dot(x3[:, p, :], w, preferred_element_type=jnp.float32) + b))
    o_ref[...] = jnp.stack(outs, axis=1).reshape(t // 8, 8 * H)


def _lin_relu_fold(x, w, b, tile, out_rows):
    n, k = x.shape
    grid = (n + tile - 1) // tile
    return pl.pallas_call(
        _lin_relu_fold_body,
        grid=(grid,),
        in_specs=[pl.BlockSpec((tile, k), lambda i: (i, 0)),
                  pl.BlockSpec((k, H), lambda i: (0, 0)),
                  pl.BlockSpec((1, H), lambda i: (0, 0))],
        out_specs=pl.BlockSpec((tile // 8, 8 * H), lambda i: (i, 0)),
        out_shape=jax.ShapeDtypeStruct((out_rows // 8, 8 * H), jnp.float32),
    )(x, w, b.reshape(1, H))


def _msg_body(e_real, ea_ref, xj_ref, wall_ref, ball_ref, brep_ref, sred_ref,
              o_ref):
    bf = TE // 8
    l = 8 * H
    qall = jnp.dot(ea_ref[...], wall_ref[...],
                   preferred_element_type=jnp.float32) + brep_ref[...]
    xjall = jnp.dot(xj_ref[...], ball_ref[...],
                    preferred_element_type=jnp.float32)
    prod = xjall * qall
    acc = jnp.dot(prod, sred_ref[...], preferred_element_type=jnp.float32)
    rows = lax.broadcasted_iota(jnp.int32, (bf, l), 0)
    lanes = lax.broadcasted_iota(jnp.int32, (bf, l), 1)
    eid = pl.program_id(0) * TE + rows * 8 + lanes // H
    o_ref[...] = jnp.where(eid < e_real, acc, 0.0)


def _msg(ea128, xj128, wall, ball, brep, sred, e, ep):
    return pl.pallas_call(
        functools.partial(_msg_body, e),
        grid=(ep // TE,),
        in_specs=[pl.BlockSpec((TE // 8, 8 * H), lambda i: (i, 0)),
                  pl.BlockSpec((TE // 8, 8 * H), lambda i: (i, 0)),
                  pl.BlockSpec((8 * H, 8 * H * H), lambda i: (0, 0)),
                  pl.BlockSpec((8 * H, 8 * H * H), lambda i: (0, 0)),
                  pl.BlockSpec((1, 8 * H * H), lambda i: (0, 0)),
                  pl.BlockSpec((8 * H * H, 8 * H), lambda i: (0, 0))],
        out_specs=pl.BlockSpec((TE // 8, 8 * H), lambda i: (i, 0)),
        out_shape=jax.ShapeDtypeStruct((ep // 8, 8 * H), jnp.float32),
    )(ea128, xj128, wall, ball, brep, sred)


def _update_body(parts_ref, xin_ref, wr_ref, br_ref, o_ref):
    nf = xin_ref.shape[0]
    x3 = xin_ref[...].reshape(nf, 8, H)
    p0 = parts_ref[0].reshape(nf, 8, H)
    p1 = parts_ref[1].reshape(nf, 8, H)
    wr = wr_ref[...]
    br = br_ref[...]
    outs = []
    for p in range(8):
        xp = x3[:, p, :]
        lin = jnp.dot(xp, wr, preferred_element_type=jnp.float32) + br
        agg = p0[:, p, :] + p1[:, p, :]
        outs.append(jax.nn.relu(agg + lin) + xp)
    o_ref[...] = jnp.stack(outs, axis=1).reshape(nf, 8 * H)


def _update(parts128, xin128, wr, br):
    nf = xin128.shape[0]
    return pl.pallas_call(
        _update_body,
        in_specs=[pl.BlockSpec(parts128.shape, lambda: (0, 0, 0)),
                  pl.BlockSpec(xin128.shape, lambda: (0, 0)),
                  pl.BlockSpec((H, H), lambda: (0, 0)),
                  pl.BlockSpec((1, H), lambda: (0, 0))],
        out_specs=pl.BlockSpec((nf, 8 * H), lambda: (0, 0)),
        out_shape=jax.ShapeDtypeStruct((nf, 8 * H), jnp.float32),
    )(parts128, xin128, wr, br.reshape(1, H))


def _pool_mlp_body(x_ref, b2_ref, w1_ref, b1_ref, w2_ref, b2b_ref,
                   w3_ref, b3b_ref, o_ref):
    nf = x_ref.shape[0]
    x3 = x_ref[...].reshape(nf, 8, H)
    p = jnp.zeros((G, H), jnp.float32)
    gids = lax.broadcasted_iota(jnp.int32, (G, nf), 0)
    for ph in range(8):
        bt = b2_ref[ph:ph + 1, :]
        mask = (bt == gids).astype(jnp.float32)
        p = p + jnp.dot(mask, x3[:, ph, :], preferred_element_type=jnp.float32)
    o1 = jax.nn.relu(jnp.dot(p, w1_ref[...],
                             preferred_element_type=jnp.float32) + b1_ref[...])
    o2 = jax.nn.relu(jnp.dot(o1, w2_ref[...],
                             preferred_element_type=jnp.float32) + b2b_ref[...])
    o_ref[...] = jnp.dot(o2, w3_ref[...],
                         preferred_element_type=jnp.float32) + b3b_ref[...]


def _pool_mlp(x128, batch, w1, b1, w2, b2, w3, b3, n):
    batch_ph = batch.reshape(n // 8, 8).T
    d1, d2, d3 = w1.shape[1], w2.shape[1], w3.shape[1]
    return pl.pallas_call(
        _pool_mlp_body,
        in_specs=[pl.BlockSpec(x128.shape, lambda: (0, 0)),
                  pl.BlockSpec((8, n // 8), lambda: (0, 0)),
                  pl.BlockSpec((H, d1), lambda: (0, 0)),
                  pl.BlockSpec((1, d1), lambda: (0, 0)),
                  pl.BlockSpec((d1, d2), lambda: (0, 0)),
                  pl.BlockSpec((1, d2), lambda: (0, 0)),
                  pl.BlockSpec((d2, d3), lambda: (0, 0)),
                  pl.BlockSpec((1, d3), lambda: (0, 0))],
        out_specs=pl.BlockSpec((G, d3), lambda: (0, 0)),
        out_shape=jax.ShapeDtypeStruct((G, d3), jnp.float32),
    )(x128, batch_ph, w1, b1.reshape(1, d1), w2, b2.reshape(1, d2),
      w3, b3.reshape(1, d3))


def _sc_gather(xin_lin, idx3, n, ep):
    epw = ep // NW
    nch = epw // CH
    mesh = plsc.VectorSubcoreMesh(core_axis_name="c", subcore_axis_name="s")

    @functools.partial(
        pl.kernel,
        out_type=jax.ShapeDtypeStruct((ep, H), jnp.float32),
        mesh=mesh,
        scratch_types=[pltpu.VMEM((nch, CH), jnp.int32),
                       pltpu.VMEM((epw, H), jnp.float32),
                       pltpu.SemaphoreType.DMA],
        compiler_params=pltpu.CompilerParams(use_tc_tiling_on_sc=False),
    )
    def k(x_hbm, idx_hbm, out_hbm, idx_v, rows_v, sem):
        wid = lax.axis_index("s") * NC + lax.axis_index("c")
        pltpu.sync_copy(idx_hbm.at[wid], idx_v)

        def fire(g, carry):
            for jj in range(GS):
                j = g * GS + jj
                pltpu.async_copy(x_hbm.at[idx_v.at[j]],
                                 rows_v.at[pl.ds(j * CH, CH)], sem)
            return carry

        def drain(g, carry):
            for jj in range(GS):
                j = g * GS + jj
                pltpu.make_async_copy(x_hbm.at[pl.ds(0, CH)],
                                      rows_v.at[pl.ds(j * CH, CH)],
                                      sem).wait()
            return carry

        lax.fori_loop(0, nch // GS, fire, 0)
        lax.fori_loop(0, nch // GS, drain, 0)
        pltpu.sync_copy(rows_v, out_hbm.at[pl.ds(wid * epw, epw)])

    return k(xin_lin, idx3)


def _sc_scatter(msg_lin, dst3, zeros_hbm, n, ep):
    epw = ep // NW
    nch = epw // CH
    rps = n // NS
    mesh = plsc.VectorSubcoreMesh(core_axis_name="c", subcore_axis_name="s")

    @functools.partial(
        pl.kernel,
        out_type=jax.ShapeDtypeStruct((NC, n, H), jnp.float32),
        mesh=mesh,
        scratch_types=[pltpu.VMEM((nch, CH), jnp.int32),
                       pltpu.VMEM((epw, H), jnp.float32),
                       pltpu.VMEM_SHARED((n, H), jnp.float32),
                       pltpu.SemaphoreType.DMA],
        compiler_params=pltpu.CompilerParams(use_tc_tiling_on_sc=False),
    )
    def k(msg_hbm, dst_hbm, z_hbm, out_hbm, dst_v, msg_v, agg_sh, sem):
        c = lax.axis_index("c")
        s = lax.axis_index("s")
        wid = s * NC + c
        pltpu.sync_copy(dst_hbm.at[wid], dst_v)
        pltpu.sync_copy(msg_hbm.at[pl.ds(wid * epw, epw)], msg_v)
        pltpu.sync_copy(z_hbm, agg_sh.at[pl.ds(s * rps, rps)])
        plsc.subcore_barrier()

        def fire(g, carry):
            for jj in range(GS):
                j = g * GS + jj
                pltpu.async_copy(msg_v.at[pl.ds(j * CH, CH)],
                                 agg_sh.at[dst_v.at[j]], sem, add=True)
            return carry

        def drain(g, carry):
            for jj in range(GS):
                j = g * GS + jj
                pltpu.make_async_copy(msg_hbm.at[pl.ds(j * CH, CH)],
                                      msg_v.at[pl.ds(j * CH, CH)],
                                      sem).wait()
            return carry

        lax.fori_loop(0, nch // GS, fire, 0)
        lax.fori_loop(0, nch // GS, drain, 0)
        plsc.subcore_barrier()
        pltpu.sync_copy(agg_sh.at[pl.ds(s * rps, rps)],
                        out_hbm.at[c].at[pl.ds(s * rps, rps)])

    return k(msg_lin, dst3, zeros_hbm)


def kernel(x, edge_index, edge_attr, batch,
           W_node, b_node, W_edge, b_edge,
           Wnn1, bnn1, Wr1, br1,
           Wnn2, bnn2, Wr2, br2,
           Wnn3, bnn3, Wr3, br3,
           W1, b1, W2, b2, W3, b3):
    n = x.shape[0]
    e = edge_index.shape[1]
    de = edge_attr.shape[1]
    ep = ((e + NW * CH - 1) // (NW * CH)) * (NW * CH)
    epw = ep // NW
    nch = epw // CH

    src = edge_index[0]
    dst = edge_index[1]
    pad = ep - e
    src3 = jnp.concatenate(
        [src, jnp.zeros((pad,), jnp.int32)]).reshape(NW, nch, CH)
    dst3 = jnp.concatenate(
        [dst, jnp.zeros((pad,), jnp.int32)]).reshape(NW, nch, CH)
    zeros_hbm = jnp.zeros((n // NS, H), jnp.float32)

    eye8 = jnp.eye(8, dtype=jnp.float32)

    def make_wall(wnn):
        w3 = wnn.reshape(H, H, H)
        wall = jnp.einsum('mho,pq->pmhqo', w3, eye8)
        return wall.reshape(8 * H, 8 * H * H)

    def make_brep(bnn):
        return jnp.tile(bnn.reshape(H, 1, H), (1, 8, 1)).reshape(1, 8 * H * H)

    ball = (jnp.einsum('pq,hk->phkq', eye8, jnp.eye(H, dtype=jnp.float32))
            [:, :, :, :, None]
            * jnp.ones((H,), jnp.float32)).reshape(8 * H, 8 * H * H)
    sred = jnp.tile(jnp.eye(8 * H, dtype=jnp.float32),
                    (H, 1))

    h128 = _lin_relu_fold(x, W_node, b_node, n, n)
    ea128 = _lin_relu_fold(edge_attr, W_edge, b_edge, TE, ep)

    xin128 = h128
    for wnn, bnn, wr, br in ((Wnn1, bnn1, Wr1, br1),
                             (Wnn2, bnn2, Wr2, br2),
                             (Wnn3, bnn3, Wr3, br3)):
        xj_lin = _sc_gather(xin128.reshape(n, H), src3, n, ep)
        msg128 = _msg(ea128, xj_lin.reshape(ep // 8, 8 * H),
                      make_wall(wnn), ball, make_brep(bnn), sred, e, ep)
        parts = _sc_scatter(msg128.reshape(ep, H), dst3, zeros_hbm, n, ep)
        xin128 = _update(parts.reshape(NC, n // 8, 8 * H), xin128, wr, br)

    return _pool_mlp(xin128, batch, W1, b1, W2, b2, W3, b3, n)


# speedup vs baseline: 1.1947x; 1.1947x over previous
import functools

import jax
import jax.numpy as jnp
from jax import lax
from jax.experimental import pallas as pl
from jax.experimental.pallas import tpu as pltpu
from jax.experimental.pallas import tpu_sc as plsc

H = 16
G = 64
NC = 2
NS = 16
NW = NC * NS
CH = 128
GS = 8

TE = 4096


def _lin_relu_fold_body(x_ref, w_ref, b_ref, o_ref):
    t, kdim = x_ref.shape
    x3 = x_ref[...].reshape(t // 8, 8, kdim)
    w = w_ref[...]
    b = b_ref[...]
    outs = []
    for p in range(8):
        outs.append(jax.nn.relu(
            jnp.dot(x3[:, p, :], w, preferred_element_type=jnp.float32) + b))
    o_ref[...] = jnp.stack(outs, axis=1).reshape(t // 8, 8 * H)


def _lin_relu_fold(x, w, b, tile, out_rows):
    n, k = x.shape
    grid = (n + tile - 1) // tile
    return pl.pallas_call(
        _lin_relu_fold_body,
        grid=(grid,),
        in_specs=[pl.BlockSpec((tile, k), lambda i: (i, 0)),
                  pl.BlockSpec((k, H), lambda i: (0, 0)),
                  pl.BlockSpec((1, H), lambda i: (0, 0))],
        out_specs=pl.BlockSpec((tile // 8, 8 * H), lambda i: (i, 0)),
        out_shape=jax.ShapeDtypeStruct((out_rows // 8, 8 * H), jnp.float32),
    )(x, w, b.reshape(1, H))


def _msg_body(e_real, ea_ref, xj_ref, wall_ref, ball_ref, brep_ref, sred_ref,
              o_ref):
    bf = TE // 8
    l = 8 * H
    qall = jnp.dot(ea_ref[...], wall_ref[...],
                   preferred_element_type=jnp.float32) + brep_ref[...]
    xjall = jnp.dot(xj_ref[...], ball_ref[...],
                    preferred_element_type=jnp.float32)
    del sred_ref
    prod = xjall * qall
    acc = None
    for h in range(H):
        t = prod[:, l * h:l * (h + 1)]
        acc = t if acc is None else acc + t
    rows = lax.broadcasted_iota(jnp.int32, (bf, l), 0)
    lanes = lax.broadcasted_iota(jnp.int32, (bf, l), 1)
    eid = pl.program_id(0) * TE + rows * 8 + lanes // H
    o_ref[...] = jnp.where(eid < e_real, acc, 0.0)


def _msg(ea128, xj128, wall, ball, brep, sred, e, ep):
    return pl.pallas_call(
        functools.partial(_msg_body, e),
        grid=(ep // TE,),
        in_specs=[pl.BlockSpec((TE // 8, 8 * H), lambda i: (i, 0)),
                  pl.BlockSpec((TE // 8, 8 * H), lambda i: (i, 0)),
                  pl.BlockSpec((8 * H, 8 * H * H), lambda i: (0, 0)),
                  pl.BlockSpec((8 * H, 8 * H * H), lambda i: (0, 0)),
                  pl.BlockSpec((1, 8 * H * H), lambda i: (0, 0)),
                  pl.BlockSpec((8 * H * H, 8 * H), lambda i: (0, 0))],
        out_specs=pl.BlockSpec((TE // 8, 8 * H), lambda i: (i, 0)),
        out_shape=jax.ShapeDtypeStruct((ep // 8, 8 * H), jnp.float32),
    )(ea128, xj128, wall, ball, brep, sred)


def _update_body(parts_ref, xin_ref, wr_ref, br_ref, o_ref):
    nf = xin_ref.shape[0]
    x3 = xin_ref[...].reshape(nf, 8, H)
    p0 = parts_ref[0].reshape(nf, 8, H)
    p1 = parts_ref[1].reshape(nf, 8, H)
    wr = wr_ref[...]
    br = br_ref[...]
    outs = []
    for p in range(8):
        xp = x3[:, p, :]
        lin = jnp.dot(xp, wr, preferred_element_type=jnp.float32) + br
        agg = p0[:, p, :] + p1[:, p, :]
        outs.append(jax.nn.relu(agg + lin) + xp)
    o_ref[...] = jnp.stack(outs, axis=1).reshape(nf, 8 * H)


def _update(parts128, xin128, wr, br):
    nf = xin128.shape[0]
    return pl.pallas_call(
        _update_body,
        in_specs=[pl.BlockSpec(parts128.shape, lambda: (0, 0, 0)),
                  pl.BlockSpec(xin128.shape, lambda: (0, 0)),
                  pl.BlockSpec((H, H), lambda: (0, 0)),
                  pl.BlockSpec((1, H), lambda: (0, 0))],
        out_specs=pl.BlockSpec((nf, 8 * H), lambda: (0, 0)),
        out_shape=jax.ShapeDtypeStruct((nf, 8 * H), jnp.float32),
    )(parts128, xin128, wr, br.reshape(1, H))


def _pool_mlp_body(x_ref, b2_ref, w1_ref, b1_ref, w2_ref, b2b_ref,
                   w3_ref, b3b_ref, o_ref):
    nf = x_ref.shape[0]
    x3 = x_ref[...].reshape(nf, 8, H)
    p = jnp.zeros((G, H), jnp.float32)
    gids = lax.broadcasted_iota(jnp.int32, (G, nf), 0)
    for ph in range(8):
        bt = b2_ref[ph:ph + 1, :]
        mask = (bt == gids).astype(jnp.float32)
        p = p + jnp.dot(mask, x3[:, ph, :], preferred_element_type=jnp.float32)
    o1 = jax.nn.relu(jnp.dot(p, w1_ref[...],
                             preferred_element_type=jnp.float32) + b1_ref[...])
    o2 = jax.nn.relu(jnp.dot(o1, w2_ref[...],
                             preferred_element_type=jnp.float32) + b2b_ref[...])
    o_ref[...] = jnp.dot(o2, w3_ref[...],
                         preferred_element_type=jnp.float32) + b3b_ref[...]


def _pool_mlp(x128, batch, w1, b1, w2, b2, w3, b3, n):
    batch_ph = batch.reshape(n // 8, 8).T
    d1, d2, d3 = w1.shape[1], w2.shape[1], w3.shape[1]
    return pl.pallas_call(
        _pool_mlp_body,
        in_specs=[pl.BlockSpec(x128.shape, lambda: (0, 0)),
                  pl.BlockSpec((8, n // 8), lambda: (0, 0)),
                  pl.BlockSpec((H, d1), lambda: (0, 0)),
                  pl.BlockSpec((1, d1), lambda: (0, 0)),
                  pl.BlockSpec((d1, d2), lambda: (0, 0)),
                  pl.BlockSpec((1, d2), lambda: (0, 0)),
                  pl.BlockSpec((d2, d3), lambda: (0, 0)),
                  pl.BlockSpec((1, d3), lambda: (0, 0))],
        out_specs=pl.BlockSpec((G, d3), lambda: (0, 0)),
        out_shape=jax.ShapeDtypeStruct((G, d3), jnp.float32),
    )(x128, batch_ph, w1, b1.reshape(1, d1), w2, b2.reshape(1, d2),
      w3, b3.reshape(1, d3))


def _sc_gather(xin_lin, idx3, n, ep):
    epw = ep // NW
    nch = epw // CH
    mesh = plsc.VectorSubcoreMesh(core_axis_name="c", subcore_axis_name="s")

    @functools.partial(
        pl.kernel,
        out_type=jax.ShapeDtypeStruct((ep, H), jnp.float32),
        mesh=mesh,
        scratch_types=[pltpu.VMEM((nch, CH), jnp.int32),
                       pltpu.VMEM((epw, H), jnp.float32),
                       pltpu.SemaphoreType.DMA],
        compiler_params=pltpu.CompilerParams(use_tc_tiling_on_sc=False),
    )
    def k(x_hbm, idx_hbm, out_hbm, idx_v, rows_v, sem):
        wid = lax.axis_index("s") * NC + lax.axis_index("c")
        pltpu.sync_copy(idx_hbm.at[wid], idx_v)

        def fire(g, carry):
            for jj in range(GS):
                j = g * GS + jj
                pltpu.async_copy(x_hbm.at[idx_v.at[j]],
                                 rows_v.at[pl.ds(j * CH, CH)], sem)
            return carry

        def drain(g, carry):
            for jj in range(GS):
                j = g * GS + jj
                pltpu.make_async_copy(x_hbm.at[pl.ds(0, CH)],
                                      rows_v.at[pl.ds(j * CH, CH)],
                                      sem).wait()
            return carry

        lax.fori_loop(0, nch // GS, fire, 0)
        lax.fori_loop(0, nch // GS, drain, 0)
        pltpu.sync_copy(rows_v, out_hbm.at[pl.ds(wid * epw, epw)])

    return k(xin_lin, idx3)


def _sc_scatter(msg_lin, dst3, zeros_hbm, n, ep):
    epw = ep // NW
    nch = epw // CH
    rps = n // NS
    mesh = plsc.VectorSubcoreMesh(core_axis_name="c", subcore_axis_name="s")

    @functools.partial(
        pl.kernel,
        out_type=jax.ShapeDtypeStruct((NC, n, H), jnp.float32),
        mesh=mesh,
        scratch_types=[pltpu.VMEM((nch, CH), jnp.int32),
                       pltpu.VMEM((epw, H), jnp.float32),
                       pltpu.VMEM_SHARED((n, H), jnp.float32),
                       pltpu.SemaphoreType.DMA],
        compiler_params=pltpu.CompilerParams(use_tc_tiling_on_sc=False),
    )
    def k(msg_hbm, dst_hbm, z_hbm, out_hbm, dst_v, msg_v, agg_sh, sem):
        c = lax.axis_index("c")
        s = lax.axis_index("s")
        wid = s * NC + c
        pltpu.sync_copy(dst_hbm.at[wid], dst_v)
        pltpu.sync_copy(msg_hbm.at[pl.ds(wid * epw, epw)], msg_v)
        pltpu.sync_copy(z_hbm, agg_sh.at[pl.ds(s * rps, rps)])
        plsc.subcore_barrier()

        def fire(g, carry):
            for jj in range(GS):
                j = g * GS + jj
                pltpu.async_copy(msg_v.at[pl.ds(j * CH, CH)],
                                 agg_sh.at[dst_v.at[j]], sem, add=True)
            return carry

        def drain(g, carry):
            for jj in range(GS):
                j = g * GS + jj
                pltpu.make_async_copy(msg_hbm.at[pl.ds(j * CH, CH)],
                                      msg_v.at[pl.ds(j * CH, CH)],
                                      sem).wait()
            return carry

        lax.fori_loop(0, nch // GS, fire, 0)
        lax.fori_loop(0, nch // GS, drain, 0)
        plsc.subcore_barrier()
        pltpu.sync_copy(agg_sh.at[pl.ds(s * rps, rps)],
                        out_hbm.at[c].at[pl.ds(s * rps, rps)])

    return k(msg_lin, dst3, zeros_hbm)


def kernel(x, edge_index, edge_attr, batch,
           W_node, b_node, W_edge, b_edge,
           Wnn1, bnn1, Wr1, br1,
           Wnn2, bnn2, Wr2, br2,
           Wnn3, bnn3, Wr3, br3,
           W1, b1, W2, b2, W3, b3):
    n = x.shape[0]
    e = edge_index.shape[1]
    de = edge_attr.shape[1]
    ep = ((e + NW * CH - 1) // (NW * CH)) * (NW * CH)
    epw = ep // NW
    nch = epw // CH

    src = edge_index[0]
    dst = edge_index[1]
    pad = ep - e
    src3 = jnp.concatenate(
        [src, jnp.zeros((pad,), jnp.int32)]).reshape(NW, nch, CH)
    dst3 = jnp.concatenate(
        [dst, jnp.zeros((pad,), jnp.int32)]).reshape(NW, nch, CH)
    zeros_hbm = jnp.zeros((n // NS, H), jnp.float32)

    eye8 = jnp.eye(8, dtype=jnp.float32)

    def make_wall(wnn):
        w3 = wnn.reshape(H, H, H)
        wall = jnp.einsum('mho,pq->pmhqo', w3, eye8)
        return wall.reshape(8 * H, 8 * H * H)

    def make_brep(bnn):
        return jnp.tile(bnn.reshape(H, 1, H), (1, 8, 1)).reshape(1, 8 * H * H)

    ball = (jnp.einsum('pq,hk->phkq', eye8, jnp.eye(H, dtype=jnp.float32))
            [:, :, :, :, None]
            * jnp.ones((H,), jnp.float32)).reshape(8 * H, 8 * H * H)
    sred = jnp.tile(jnp.eye(8 * H, dtype=jnp.float32),
                    (H, 1))

    h128 = _lin_relu_fold(x, W_node, b_node, n, n)
    ea128 = _lin_relu_fold(edge_attr, W_edge, b_edge, TE, ep)

    xin128 = h128
    for wnn, bnn, wr, br in ((Wnn1, bnn1, Wr1, br1),
                             (Wnn2, bnn2, Wr2, br2),
                             (Wnn3, bnn3, Wr3, br3)):
        xj_lin = _sc_gather(xin128.reshape(n, H), src3, n, ep)
        msg128 = _msg(ea128, xj_lin.reshape(ep // 8, 8 * H),
                      make_wall(wnn), ball, make_brep(bnn), sred, e, ep)
        parts = _sc_scatter(msg128.reshape(ep, H), dst3, zeros_hbm, n, ep)
        xin128 = _update(parts.reshape(NC, n // 8, 8 * H), xin128, wr, br)

    return _pool_mlp(xin128, batch, W1, b1, W2, b2, W3, b3, n)
